# baseline (speedup 1.0000x reference)
import functools
import math

import jax
import jax.numpy as jnp
from jax import lax
from jax.experimental import pallas as pl
from jax.experimental.pallas import tpu as pltpu
from jax.experimental.pallas import tpu_sc as plsc

VOCAB_SIZE = 1000000
D = 64
SCALE = math.sqrt(D)

NC = 2
NS = 16
NW = NC * NS

I, J = 4096, 200
IT, IL = I // 128, 128
JT, JS = J // 8, 8
G = IT * JT * JS
GPW = G // NW
PPW = GPW // 2

NR = 4


def _embed_body(idx_hbm, table_hbm, out_hbm,
                idx_v, r0, r1, r2, r3, rm0, rm1,
                semg0, semg1, semg2, semg3, sems0, sems1):
    rbufs = (r0, r1, r2, r3)
    rms = (rm0, rm1)
    gsems = (semg0, semg1, semg2, semg3)
    ssems = (sems0, sems1)

    wid = lax.axis_index("s") * NC + lax.axis_index("c")
    base_g = wid * GPW

    pltpu.sync_copy(idx_hbm.at[pl.ds(base_g * 128, GPW * 128)], idx_v)

    def start_gather(p, a):
        pltpu.async_copy(
            table_hbm.at[idx_v.at[pl.ds(p * 256, 256)]], rbufs[a], gsems[a])

    def wait_gather(a):
        pltpu.make_async_copy(
            table_hbm.at[pl.ds(0, 256)], rbufs[a], gsems[a]).wait()

    def wait_stores(s):
        pltpu.make_async_copy(
            rms[s].at[:, :, pl.ds(0, IL)],
            out_hbm.at[pl.ds(0, 8), 0], ssems[s]).wait()

    iota = lax.iota(jnp.int32, 16)

    start_gather(0, 0)
    start_gather(1, 1)
    start_gather(2, 2)

    @pl.loop(0, PPW, step=NR)
    def _(p0):
        for a in range(NR):
            p = p0 + a

            wait_gather(a)
            rb = rbufs[a]

            for s in range(2):
                rbh = rb.at[pl.ds(s * 128, 128)]
                rmx = rms[s]

                @pl.when(p >= 1)
                def _():
                    wait_stores(s)

                @plsc.parallel_loop(0, 128, unroll=16)
                def _(il):
                    colv = jnp.full((16,), il, jnp.int32)
                    vals = [rbh[il, pl.ds(q * 16, 16)] for q in range(4)]
                    for q in range(4):
                        dv = iota + q * 16
                        plsc.store_scatter(
                            rmx, [dv >> 3, dv & 7, colv], vals[q] * SCALE)

                g = base_g + 2 * p + s
                jt = g >> 8
                it = (g >> 3) & 31
                js = g & 7
                row_base = jt * 64 + js * 8
                pltpu.async_copy(
                    rmx.at[:, :, pl.ds(0, IL)],
                    out_hbm.at[pl.ds(row_base, 8), it], ssems[s])

            @pl.when(p + 3 < PPW)
            def _():
                start_gather(p + 3, (a + 3) % NR)

    wait_stores(0)
    wait_stores(1)


def kernel(x, table):
    assert x.shape == (I, J) and table.shape == (VOCAB_SIZE, D)
    xp = (x.astype(jnp.int32)
          .reshape(IT, IL, JT, JS)
          .transpose(2, 0, 3, 1)
          .reshape(-1))

    mesh = plsc.VectorSubcoreMesh(core_axis_name="c", subcore_axis_name="s")
    out4 = pl.kernel(
        _embed_body,
        out_type=jax.ShapeDtypeStruct((J * 8, IT, 8, IL), jnp.float32),
        mesh=mesh,
        compiler_params=pltpu.CompilerParams(
            use_tc_tiling_on_sc=False, needs_layout_passes=False),
        scratch_types=[
            pltpu.VMEM((GPW * 128,), jnp.int32),
            pltpu.VMEM((256, D), jnp.float32),
            pltpu.VMEM((256, D), jnp.float32),
            pltpu.VMEM((256, D), jnp.float32),
            pltpu.VMEM((256, D), jnp.float32),
            pltpu.VMEM((8, 8, 136), jnp.float32),
            pltpu.VMEM((8, 8, 136), jnp.float32),
            pltpu.SemaphoreType.DMA,
            pltpu.SemaphoreType.DMA,
            pltpu.SemaphoreType.DMA,
            pltpu.SemaphoreType.DMA,
            pltpu.SemaphoreType.DMA,
            pltpu.SemaphoreType.DMA,
        ],
    )(xp, table)
    return (out4.reshape(J, 8, IT, 8, IL)
            .transpose(2, 4, 0, 1, 3)
            .reshape(I, J, D))


# speedup vs baseline: 1.0071x; 1.0071x over previous
import functools
import math

import jax
import jax.numpy as jnp
from jax import lax
from jax.experimental import pallas as pl
from jax.experimental.pallas import tpu as pltpu
from jax.experimental.pallas import tpu_sc as plsc

VOCAB_SIZE = 1000000
D = 64
SCALE = math.sqrt(D)

NC = 2
NS = 16
NW = NC * NS

I, J = 4096, 200
IT, IL = I // 128, 128
JT, JS = J // 8, 8
G = IT * JT * JS
GPW = G // NW
PPW = GPW // 2

NR = 4


def _embed_body(idx_hbm, table_hbm, out_hbm,
                idx_v, r0, r1, r2, r3, rm0, rm1,
                semg0, semg1, semg2, semg3, sems0, sems1):
    rbufs = (r0, r1, r2, r3)
    rms = (rm0, rm1)
    gsems = (semg0, semg1, semg2, semg3)
    ssems = (sems0, sems1)

    wid = lax.axis_index("s") * NC + lax.axis_index("c")
    base_g = wid * GPW

    pltpu.sync_copy(idx_hbm.at[pl.ds(base_g * 128, GPW * 128)], idx_v)

    def start_gather(p, a):
        pltpu.async_copy(
            table_hbm.at[idx_v.at[pl.ds(p * 256, 256)]], rbufs[a], gsems[a])

    def wait_gather(a):
        pltpu.make_async_copy(
            table_hbm.at[pl.ds(0, 256)], rbufs[a], gsems[a]).wait()

    def wait_stores(s):
        pltpu.make_async_copy(
            rms[s].at[:, :, pl.ds(0, IL)],
            out_hbm.at[pl.ds(0, 8), 0], ssems[s]).wait()

    iota = lax.iota(jnp.int32, 16)

    start_gather(0, 0)
    start_gather(1, 1)
    start_gather(2, 2)

    @pl.loop(0, PPW, step=NR)
    def _(p0):
        for a in range(NR):
            p = p0 + a

            wait_gather(a)
            rb = rbufs[a]

            for s in range(2):
                rbh = rb.at[pl.ds(s * 128, 128)]
                rmx = rms[s]

                @pl.when(p >= 1)
                def _():
                    wait_stores(s)

                @plsc.parallel_loop(0, 128, unroll=8)
                def _(il):
                    colv = jnp.full((16,), il, jnp.int32)
                    vals = [rbh[il, pl.ds(q * 16, 16)] for q in range(4)]
                    for q in range(4):
                        dv = iota + q * 16
                        plsc.store_scatter(
                            rmx, [dv >> 3, dv & 7, colv], vals[q] * SCALE)

                g = base_g + 2 * p + s
                jt = g >> 8
                it = (g >> 3) & 31
                js = g & 7
                row_base = jt * 64 + js * 8
                pltpu.async_copy(
                    rmx.at[:, :, pl.ds(0, IL)],
                    out_hbm.at[pl.ds(row_base, 8), it], ssems[s])

            @pl.when(p + 3 < PPW)
            def _():
                start_gather(p + 3, (a + 3) % NR)

    wait_stores(0)
    wait_stores(1)


def kernel(x, table):
    assert x.shape == (I, J) and table.shape == (VOCAB_SIZE, D)
    xp = (x.astype(jnp.int32)
          .reshape(IT, IL, JT, JS)
          .transpose(2, 0, 3, 1)
          .reshape(-1))

    mesh = plsc.VectorSubcoreMesh(core_axis_name="c", subcore_axis_name="s")
    out4 = pl.kernel(
        _embed_body,
        out_type=jax.ShapeDtypeStruct((J * 8, IT, 8, IL), jnp.float32),
        mesh=mesh,
        compiler_params=pltpu.CompilerParams(
            use_tc_tiling_on_sc=False, needs_layout_passes=False),
        scratch_types=[
            pltpu.VMEM((GPW * 128,), jnp.int32),
            pltpu.VMEM((256, D), jnp.float32),
            pltpu.VMEM((256, D), jnp.float32),
            pltpu.VMEM((256, D), jnp.float32),
            pltpu.VMEM((256, D), jnp.float32),
            pltpu.VMEM((8, 8, 136), jnp.float32),
            pltpu.VMEM((8, 8, 136), jnp.float32),
            pltpu.SemaphoreType.DMA,
            pltpu.SemaphoreType.DMA,
            pltpu.SemaphoreType.DMA,
            pltpu.SemaphoreType.DMA,
            pltpu.SemaphoreType.DMA,
            pltpu.SemaphoreType.DMA,
        ],
    )(xp, table)
    return (out4.reshape(J, 8, IT, 8, IL)
            .transpose(2, 4, 0, 1, 3)
            .reshape(I, J, D))
